# Initial kernel scaffold; baseline (speedup 1.0000x reference)
import jax
import jax.numpy as jnp
from jax.experimental import pallas as pl


def kernel(H, V, Z, layer_i, H_2d, mask_2d, E_2d_index, E_2d_attr, Z_3d, mask_3d, E_dist_index, E_dist_val, virtual_node_embed, edge_type_table, W_rbf, W_edge2d, W_i, W1, b1, W2, b2, W_h, W_a, W_g):
    raise NotImplementedError("write your pallas kernel here")


# speedup vs baseline: 1.3106x; 1.3106x over previous
import functools
import jax
import jax.numpy as jnp
import numpy as np
from jax.experimental import pallas as pl
from jax.experimental.pallas import tpu as pltpu

CUTOFF = 6.0
NRBF = 64


def _silu(x):
    return x * jax.nn.sigmoid(x)


def _rbf(d_col, tile):
    centers = jax.lax.broadcasted_iota(jnp.int32, (tile, NRBF), 1).astype(
        jnp.float32) * (CUTOFF / (NRBF - 1))
    width = CUTOFF / NRBF
    gamma = 1.0 / (2.0 * width * width)
    env = 0.5 * (jnp.cos(jnp.pi * jnp.clip(d_col / CUTOFF, 0.0, 1.0)) + 1.0)
    return jnp.exp(-gamma * (d_col - centers) ** 2) * env


def _nodeproj_body(h_ref, h2d_ref, vn_ref, wi_ref, w1_ref,
                   ps_ref, pd_ref, psv_ref):
    f32 = jnp.float32
    a = jnp.dot(h_ref[...], wi_ref[0:128, :], preferred_element_type=f32)
    br = jnp.dot(h2d_ref[...], wi_ref[128:256, :], preferred_element_type=f32)
    cvec = jnp.dot(vn_ref[...], wi_ref[128:256, :], preferred_element_type=f32)
    nr = a + br
    nv = a + cvec
    w1s = w1_ref[0:128, :]
    w1d = w1_ref[128:256, :]
    ps_ref[...] = jnp.dot(nr, w1s, preferred_element_type=f32)
    pd_ref[...] = jnp.dot(nr, w1d, preferred_element_type=f32)
    psv_ref[...] = jnp.dot(nv, w1s, preferred_element_type=f32)


def _dense3d_body(psv_ref, pd_ref, geo_ref, et_ref, w1_ref, b1_ref,
                  w2_ref, b2_ref, wa_ref, m3_ref, v3_ref):
    f32 = jnp.float32
    tile = psv_ref.shape[0]
    d = geo_ref[:, 0:1]
    rbf = _rbf(d, tile)
    bc1 = b1_ref[...] + jnp.dot(et_ref[1:2, :], w1_ref[320:336, :],
                                preferred_element_type=f32)
    pre = psv_ref[...] + pd_ref[...] + jnp.dot(
        rbf, w1_ref[336:400, :], preferred_element_type=f32) + bc1
    m1 = _silu(pre)
    m = _silu(jnp.dot(m1, w2_ref[...], preferred_element_type=f32)
              + b2_ref[...])
    msk = geo_ref[:, 4:5]
    m3_ref[...] = m * msk
    a = jnp.dot(m, wa_ref[...], preferred_element_type=f32) * msk
    for k in range(3):
        v3_ref[:, k, :] = a * geo_ref[:, 1 + k:2 + k]


def _edge_body(n2tiles, g_ref, geo_ref, attr_ref, vs_ref, et_ref, w_e2d_ref,
               w_rbf_ref, w1_ref, b1_ref, w2_ref, b2_ref, wa_ref, wg_ref,
               m_ref, vec_ref):
    f32 = jnp.float32
    tile = g_ref.shape[0]
    i = pl.program_id(0)
    w1ea = w1_ref[256:320, :]
    bc = b1_ref[...] + jnp.dot(
        jnp.where(i < n2tiles, et_ref[0:1, :], et_ref[2:3, :]),
        w1_ref[320:336, :], preferred_element_type=f32)
    m2d = jnp.dot(w_e2d_ref[...], w1ea, preferred_element_type=f32)
    t2 = jnp.dot(attr_ref[...], m2d, preferred_element_type=f32)
    mdist = jnp.dot(w_rbf_ref[...], w1ea, preferred_element_type=f32)
    rbv = _rbf(geo_ref[:, 4:5], tile)
    td = jnp.dot(rbv, mdist, preferred_element_type=f32)
    t = jnp.where(i < n2tiles, t2, td) + bc
    d = geo_ref[:, 0:1]
    rbf = _rbf(d, tile)
    pre = g_ref[...] + t + jnp.dot(rbf, w1_ref[336:400, :],
                                   preferred_element_type=f32)
    m1 = _silu(pre)
    m = _silu(jnp.dot(m1, w2_ref[...], preferred_element_type=f32)
              + b2_ref[...])
    m_ref[...] = m
    a = jnp.dot(m, wa_ref[...], preferred_element_type=f32)
    g = jnp.dot(m, wg_ref[...], preferred_element_type=f32)
    for k in range(3):
        vec_ref[:, k, :] = a * geo_ref[:, 1 + k:2 + k] + g * vs_ref[:, k, :]


def _final_body(agg_ref, vsum_ref, wh_ref, um_ref, h_ref, v_ref):
    f32 = jnp.float32
    um = um_ref[:, 0:1]
    h = jnp.dot(agg_ref[...], wh_ref[...], preferred_element_type=f32)
    h_ref[...] = jnp.clip(h, -100.0, 100.0) * um
    for k in range(3):
        v_ref[:, k, :] = jnp.clip(vsum_ref[:, k, :], -100.0, 100.0) * um


def _w(spec_shape):
    return pl.BlockSpec(spec_shape, lambda i: tuple(0 for _ in spec_shape))


def kernel(H, V, Z, layer_i, H_2d, mask_2d, E_2d_index, E_2d_attr, Z_3d,
           mask_3d, E_dist_index, E_dist_val, virtual_node_embed,
           edge_type_table, W_rbf, W_edge2d, W_i, W1, b1, W2, b2, W_h,
           W_a, W_g):
    del layer_i
    f32 = jnp.float32
    N = H.shape[0]
    E1 = E_2d_index.shape[1]
    E2 = E_dist_index.shape[1]
    H1 = H.shape[1]

    b1r = b1.reshape(1, H1)
    b2r = b2.reshape(1, H1)

    NT = 2000
    ps, pd, psv = pl.pallas_call(
        _nodeproj_body,
        grid=(N // NT,),
        in_specs=[
            pl.BlockSpec((NT, H1), lambda i: (i, 0)),
            pl.BlockSpec((NT, H1), lambda i: (i, 0)),
            _w((1, H1)), _w((2 * H1, H1)), _w((400, H1)),
        ],
        out_specs=[pl.BlockSpec((NT, H1), lambda i: (i, 0))] * 3,
        out_shape=[jax.ShapeDtypeStruct((N, H1), f32)] * 3,
    )(H, H_2d, virtual_node_embed, W_i, W1)

    rel3 = Z_3d - Z
    d3 = jnp.sqrt(jnp.sum(rel3 * rel3, axis=-1) + 1e-8)
    ru3 = rel3 / (d3[:, None] + 1.0)
    m3f = mask_3d.astype(f32)
    geo3 = jnp.concatenate(
        [d3[:, None], ru3, m3f[:, None], jnp.zeros((N, 3), f32)], axis=1)
    m3, v3 = pl.pallas_call(
        _dense3d_body,
        grid=(N // NT,),
        in_specs=[
            pl.BlockSpec((NT, H1), lambda i: (i, 0)),
            pl.BlockSpec((NT, H1), lambda i: (i, 0)),
            pl.BlockSpec((NT, 8), lambda i: (i, 0)),
            _w((3, 16)), _w((400, H1)), _w((1, H1)),
            _w((H1, H1)), _w((1, H1)), _w((H1, H1)),
        ],
        out_specs=[pl.BlockSpec((NT, H1), lambda i: (i, 0)),
                   pl.BlockSpec((NT, 3, H1), lambda i: (i, 0, 0))],
        out_shape=[jax.ShapeDtypeStruct((N, H1), f32),
                   jax.ShapeDtypeStruct((N, 3, H1), f32)],
    )(psv, pd, geo3, edge_type_table, W1, b1r, W2, b2r, W_a)

    ET = 256
    e1p = (E1 + ET - 1) // ET * ET
    e2p = (E2 + ET - 1) // ET * ET
    ep = e1p + e2p
    n2tiles = e1p // ET

    def pad_idx(x, n, fill):
        return jnp.concatenate([x, jnp.full((n - x.shape[0],), fill, x.dtype)])

    src = jnp.concatenate([pad_idx(E_2d_index[0], e1p, 0),
                           pad_idx(E_dist_index[0], ep - e1p, 0)])
    dst_g = jnp.concatenate([pad_idx(E_2d_index[1], e1p, 0),
                             pad_idx(E_dist_index[1], ep - e1p, 0)])
    dst_s = jnp.concatenate([pad_idx(E_2d_index[1], e1p, N),
                             pad_idx(E_dist_index[1], ep - e1p, N)])

    gsum = jnp.take(ps, src, axis=0) + jnp.take(pd, dst_g, axis=0)
    zs = jnp.take(Z, src, axis=0)
    zd = jnp.take(Z, dst_g, axis=0)
    rel = zs - zd
    d = jnp.sqrt(jnp.sum(rel * rel, axis=-1) + 1e-8)
    ru = rel / (d[:, None] + 1.0)
    dval = jnp.concatenate([jnp.zeros((e1p,), f32),
                            pad_idx(E_dist_val, ep - e1p, 0.0)])
    geo = jnp.concatenate(
        [d[:, None], ru, dval[:, None], jnp.zeros((ep, 3), f32)], axis=1)
    attr = jnp.zeros((ep, 16), f32).at[:E1].set(E_2d_attr.T)
    vt = jnp.transpose(V, (0, 2, 1))
    vs = jnp.take(vt, src, axis=0)

    m_e, vec_e = pl.pallas_call(
        functools.partial(_edge_body, n2tiles),
        grid=(ep // ET,),
        in_specs=[
            pl.BlockSpec((ET, H1), lambda i: (i, 0)),
            pl.BlockSpec((ET, 8), lambda i: (i, 0)),
            pl.BlockSpec((ET, 16), lambda i: (i, 0)),
            pl.BlockSpec((ET, 3, H1), lambda i: (i, 0, 0)),
            _w((3, 16)), _w((16, 64)), _w((64, 64)), _w((400, H1)),
            _w((1, H1)), _w((H1, H1)), _w((1, H1)), _w((H1, H1)),
            _w((H1, H1)),
        ],
        out_specs=[pl.BlockSpec((ET, H1), lambda i: (i, 0)),
                   pl.BlockSpec((ET, 3, H1), lambda i: (i, 0, 0))],
        out_shape=[jax.ShapeDtypeStruct((ep, H1), f32),
                   jax.ShapeDtypeStruct((ep, 3, H1), f32)],
    )(gsum, geo, attr, vs, edge_type_table, W_edge2d, W_rbf, W1, b1r,
      W2, b2r, W_a, W_g)

    agg = jnp.zeros((N + 8, H1), f32).at[dst_s].add(m_e)[:N] + m3
    vsum = jnp.zeros((N + 8, 3, H1), f32).at[dst_s].add(vec_e)[:N] + v3

    mask_dist = (jnp.zeros((N,), bool).at[E_dist_index[0]].set(True)
                 .at[E_dist_index[1]].set(True))
    um = (mask_2d | mask_3d | mask_dist).astype(f32)
    um8 = jnp.broadcast_to(um[:, None], (N, 8))

    h_add, v_out = pl.pallas_call(
        _final_body,
        grid=(N // NT,),
        in_specs=[
            pl.BlockSpec((NT, H1), lambda i: (i, 0)),
            pl.BlockSpec((NT, 3, H1), lambda i: (i, 0, 0)),
            _w((H1, H1)),
            pl.BlockSpec((NT, 8), lambda i: (i, 0)),
        ],
        out_specs=[pl.BlockSpec((NT, H1), lambda i: (i, 0)),
                   pl.BlockSpec((NT, 3, H1), lambda i: (i, 0, 0))],
        out_shape=[jax.ShapeDtypeStruct((N, H1), f32),
                   jax.ShapeDtypeStruct((N, 3, H1), f32)],
    )(agg, vsum, W_h, um8)

    return (h_add, jnp.transpose(v_out, (0, 2, 1)))


# speedup vs baseline: 5.5349x; 4.2231x over previous
import functools
import jax
import jax.numpy as jnp
import numpy as np
from jax.experimental import pallas as pl
from jax.experimental.pallas import tpu as pltpu
from jax.experimental.pallas import tpu_sc as plsc

CUTOFF = 6.0
NRBF = 64


def _silu(x):
    return x * jax.nn.sigmoid(x)


def _rbf(d_col, tile):
    centers = jax.lax.broadcasted_iota(jnp.int32, (tile, NRBF), 1).astype(
        jnp.float32) * (CUTOFF / (NRBF - 1))
    width = CUTOFF / NRBF
    gamma = 1.0 / (2.0 * width * width)
    env = 0.5 * (jnp.cos(jnp.pi * jnp.clip(d_col / CUTOFF, 0.0, 1.0)) + 1.0)
    return jnp.exp(-gamma * (d_col - centers) ** 2) * env


def _nodeproj_body(h_ref, h2d_ref, vn_ref, wi_ref, w1_ref,
                   ps_ref, pd_ref, psv_ref):
    f32 = jnp.float32
    a = jnp.dot(h_ref[...], wi_ref[0:128, :], preferred_element_type=f32)
    br = jnp.dot(h2d_ref[...], wi_ref[128:256, :], preferred_element_type=f32)
    cvec = jnp.dot(vn_ref[...], wi_ref[128:256, :], preferred_element_type=f32)
    nr = a + br
    nv = a + cvec
    w1s = w1_ref[0:128, :]
    w1d = w1_ref[128:256, :]
    ps_ref[...] = jnp.dot(nr, w1s, preferred_element_type=f32)
    pd_ref[...] = jnp.dot(nr, w1d, preferred_element_type=f32)
    psv_ref[...] = jnp.dot(nv, w1s, preferred_element_type=f32)


def _dense3d_body(psv_ref, pd_ref, geo_ref, et_ref, w1_ref, b1_ref,
                  w2_ref, b2_ref, wa_ref, m3_ref, v3_ref):
    f32 = jnp.float32
    tile = psv_ref.shape[0]
    d = geo_ref[:, 0:1]
    rbf = _rbf(d, tile)
    bc1 = b1_ref[...] + jnp.dot(et_ref[1:2, :], w1_ref[320:336, :],
                                preferred_element_type=f32)
    pre = psv_ref[...] + pd_ref[...] + jnp.dot(
        rbf, w1_ref[336:400, :], preferred_element_type=f32) + bc1
    m1 = _silu(pre)
    m = _silu(jnp.dot(m1, w2_ref[...], preferred_element_type=f32)
              + b2_ref[...])
    msk = geo_ref[:, 4:5]
    m3_ref[...] = m * msk
    a = jnp.dot(m, wa_ref[...], preferred_element_type=f32) * msk
    for k in range(3):
        v3_ref[:, pl.ds(k * 128, 128)] = a * geo_ref[:, 1 + k:2 + k]


def _edge_body(n2tiles, g_ref, geo_ref, attr_ref, vs_ref, et_ref, w_e2d_ref,
               w_rbf_ref, w1_ref, b1_ref, w2_ref, b2_ref, wa_ref, wg_ref,
               m_ref, *vec_refs):
    f32 = jnp.float32
    tile = g_ref.shape[0]
    i = pl.program_id(0)
    w1ea = w1_ref[256:320, :]
    bc = b1_ref[...] + jnp.dot(
        jnp.where(i < n2tiles, et_ref[0:1, :], et_ref[2:3, :]),
        w1_ref[320:336, :], preferred_element_type=f32)
    m2d = jnp.dot(w_e2d_ref[...], w1ea, preferred_element_type=f32)
    t2 = jnp.dot(attr_ref[...], m2d, preferred_element_type=f32)
    mdist = jnp.dot(w_rbf_ref[...], w1ea, preferred_element_type=f32)
    rbv = _rbf(geo_ref[:, 4:5], tile)
    td = jnp.dot(rbv, mdist, preferred_element_type=f32)
    t = jnp.where(i < n2tiles, t2, td) + bc
    d = geo_ref[:, 0:1]
    rbf = _rbf(d, tile)
    pre = g_ref[...] + t + jnp.dot(rbf, w1_ref[336:400, :],
                                   preferred_element_type=f32)
    m1 = _silu(pre)
    m = _silu(jnp.dot(m1, w2_ref[...], preferred_element_type=f32)
              + b2_ref[...])
    m_ref[...] = m
    a = jnp.dot(m, wa_ref[...], preferred_element_type=f32)
    g = jnp.dot(m, wg_ref[...], preferred_element_type=f32)
    for k, vref in enumerate(vec_refs):
        vref[...] = (a * geo_ref[:, 1 + k:2 + k]
                     + g * vs_ref[:, pl.ds(k * 128, 128)])


def _final_body(agg_ref, vsum_ref, wh_ref, um_ref, h_ref, v_ref):
    f32 = jnp.float32
    um = um_ref[:, 0:1]
    h = jnp.dot(agg_ref[...], wh_ref[...], preferred_element_type=f32)
    h_ref[...] = jnp.clip(h, -100.0, 100.0) * um
    v_ref[...] = jnp.clip(vsum_ref[...], -100.0, 100.0) * um


def _make_sc_scatter(E_s, D, B, NB, NPC, R):
    mesh = plsc.VectorSubcoreMesh(core_axis_name="c", subcore_axis_name="s",
                                  num_cores=2, num_subcores=16)
    Z = R // 16
    ZC = 32
    nzch = Z // ZC
    nj = B // 128
    k16 = D // 16
    chunk = B * NB
    crows = chunk // 128

    def body(data_hbm, dst_hbm, out_hbm, databuf, idxbuf, zbuf, acc):
        c = jax.lax.axis_index("c")
        s = jax.lax.axis_index("s")

        def zstore(i, car):
            r = i // k16
            col = (i % k16) * 16
            zbuf[r, pl.ds(col, 16)] = jnp.zeros((16,), jnp.float32)
            return car
        jax.lax.fori_loop(0, ZC * k16, zstore, 0)
        for t in range(nzch):
            off = pl.multiple_of(s * Z + t * ZC, ZC)
            pltpu.sync_copy(zbuf, acc.at[pl.ds(off, ZC)])
        plsc.subcore_barrier()

        ibase = pl.multiple_of(s * crows, 8)
        pltpu.sync_copy(dst_hbm.at[pl.ds(ibase, crows)], idxbuf)
        lo = c * NPC

        def fix(i, car2):
            j = i // 8
            col = (i % 8) * 16
            v = idxbuf[j, pl.ds(col, 16)]
            loc = v - lo
            ok = (loc >= 0) & (loc < NPC)
            idxbuf[j, pl.ds(col, 16)] = jnp.where(ok, loc, NPC)
            return car2
        jax.lax.fori_loop(0, crows * 8, fix, 0)

        def block(b, car):
            base = pl.multiple_of(s * chunk + b * B, 8)
            pltpu.sync_copy(data_hbm.at[pl.ds(base, B)], databuf)
            for j in range(nj):
                pltpu.sync_copy(databuf.at[pl.ds(j * 128, 128)],
                                acc.at[idxbuf.at[b * nj + j]], add=True)
            return car
        jax.lax.fori_loop(0, NB, block, 0)
        plsc.subcore_barrier()
        for t in range(nzch):
            off = pl.multiple_of(s * Z + t * ZC, ZC)
            pltpu.sync_copy(acc.at[pl.ds(off, ZC)],
                            out_hbm.at[c, pl.ds(off, ZC)])

    f32 = jnp.float32
    return pl.kernel(
        body,
        out_type=jax.ShapeDtypeStruct((2, R, D), f32),
        mesh=mesh,
        scratch_types=[pltpu.VMEM((B, D), f32),
                       pltpu.VMEM((crows, 128), jnp.int32),
                       pltpu.VMEM((ZC, D), f32),
                       pltpu.VMEM_SHARED((R, D), f32)],
    )


def _w(spec_shape):
    return pl.BlockSpec(spec_shape, lambda i: tuple(0 for _ in spec_shape))


def kernel(H, V, Z, layer_i, H_2d, mask_2d, E_2d_index, E_2d_attr, Z_3d,
           mask_3d, E_dist_index, E_dist_val, virtual_node_embed,
           edge_type_table, W_rbf, W_edge2d, W_i, W1, b1, W2, b2, W_h,
           W_a, W_g):
    del layer_i
    f32 = jnp.float32
    N = H.shape[0]
    E1 = E_2d_index.shape[1]
    E2 = E_dist_index.shape[1]
    H1 = H.shape[1]

    b1r = b1.reshape(1, H1)
    b2r = b2.reshape(1, H1)

    NT = 2000
    ps, pd, psv = pl.pallas_call(
        _nodeproj_body,
        grid=(N // NT,),
        in_specs=[
            pl.BlockSpec((NT, H1), lambda i: (i, 0)),
            pl.BlockSpec((NT, H1), lambda i: (i, 0)),
            _w((1, H1)), _w((2 * H1, H1)), _w((400, H1)),
        ],
        out_specs=[pl.BlockSpec((NT, H1), lambda i: (i, 0))] * 3,
        out_shape=[jax.ShapeDtypeStruct((N, H1), f32)] * 3,
    )(H, H_2d, virtual_node_embed, W_i, W1)

    rel3 = Z_3d - Z
    d3 = jnp.sqrt(jnp.sum(rel3 * rel3, axis=-1) + 1e-8)
    ru3 = rel3 / (d3[:, None] + 1.0)
    m3f = mask_3d.astype(f32)
    geo3 = jnp.concatenate(
        [d3[:, None], ru3, m3f[:, None], jnp.zeros((N, 3), f32)], axis=1)
    m3, v3 = pl.pallas_call(
        _dense3d_body,
        grid=(N // NT,),
        in_specs=[
            pl.BlockSpec((NT, H1), lambda i: (i, 0)),
            pl.BlockSpec((NT, H1), lambda i: (i, 0)),
            pl.BlockSpec((NT, 8), lambda i: (i, 0)),
            _w((3, 16)), _w((400, H1)), _w((1, H1)),
            _w((H1, H1)), _w((1, H1)), _w((H1, H1)),
        ],
        out_specs=[pl.BlockSpec((NT, H1), lambda i: (i, 0)),
                   pl.BlockSpec((NT, 3 * H1), lambda i: (i, 0))],
        out_shape=[jax.ShapeDtypeStruct((N, H1), f32),
                   jax.ShapeDtypeStruct((N, 3 * H1), f32)],
    )(psv, pd, geo3, edge_type_table, W1, b1r, W2, b2r, W_a)

    ET = 256
    e1p = (E1 + ET - 1) // ET * ET
    ep = ((e1p + E2 + 8191) // 8192) * 8192
    n2tiles = e1p // ET

    def pad_idx(x, n, fill):
        return jnp.concatenate([x, jnp.full((n - x.shape[0],), fill, x.dtype)])

    src = jnp.concatenate([pad_idx(E_2d_index[0], e1p, 0),
                           pad_idx(E_dist_index[0], ep - e1p, 0)])
    dst_g = jnp.concatenate([pad_idx(E_2d_index[1], e1p, 0),
                             pad_idx(E_dist_index[1], ep - e1p, 0)])
    dst_s = jnp.concatenate([pad_idx(E_2d_index[1], e1p, N),
                             pad_idx(E_dist_index[1], ep - e1p, N)])

    gsum = jnp.take(ps, src, axis=0) + jnp.take(pd, dst_g, axis=0)
    zs = jnp.take(Z, src, axis=0)
    zd = jnp.take(Z, dst_g, axis=0)
    rel = zs - zd
    d = jnp.sqrt(jnp.sum(rel * rel, axis=-1) + 1e-8)
    ru = rel / (d[:, None] + 1.0)
    dval = jnp.concatenate([jnp.zeros((e1p,), f32),
                            pad_idx(E_dist_val, ep - e1p, 0.0)])
    geo = jnp.concatenate(
        [d[:, None], ru, dval[:, None], jnp.zeros((ep, 3), f32)], axis=1)
    attr = jnp.zeros((ep, 16), f32).at[:E1].set(E_2d_attr.T)
    vt = jnp.transpose(V, (0, 2, 1)).reshape(N, 3 * H1)
    vs = jnp.take(vt, src, axis=0)

    m_e = pl.pallas_call(
        functools.partial(_edge_body, n2tiles),
        grid=(ep // ET,),
        in_specs=[
            pl.BlockSpec((ET, H1), lambda i: (i, 0)),
            pl.BlockSpec((ET, 8), lambda i: (i, 0)),
            pl.BlockSpec((ET, 16), lambda i: (i, 0)),
            pl.BlockSpec((ET, 3 * H1), lambda i: (i, 0)),
            _w((3, 16)), _w((16, 64)), _w((64, 64)), _w((400, H1)),
            _w((1, H1)), _w((H1, H1)), _w((1, H1)), _w((H1, H1)),
            _w((H1, H1)),
        ],
        out_specs=[pl.BlockSpec((ET, H1), lambda i: (i, 0))] * 4,
        out_shape=[jax.ShapeDtypeStruct((ep, H1), f32)] * 4,
    )(gsum, geo, attr, vs, edge_type_table, W_edge2d, W_rbf, W1, b1r,
      W2, b2r, W_a, W_g)
    m_e, vx_e, vy_e, vz_e = m_e

    NPC = N // 2
    R = ((NPC + 1 + 1023) // 1024) * 1024
    dst2d = dst_s.reshape(ep // 128, 128)
    scat = _make_sc_scatter(ep, H1, 512, ep // (16 * 512), NPC, R)

    def seg(x):
        o = scat(x, dst2d)
        return jnp.concatenate([o[0, :NPC], o[1, :NPC]])

    agg = seg(m_e) + m3
    vsum = jnp.concatenate([seg(vx_e), seg(vy_e), seg(vz_e)], axis=1) + v3

    mask_dist = (jnp.zeros((N,), bool).at[E_dist_index[0]].set(True)
                 .at[E_dist_index[1]].set(True))
    um = (mask_2d | mask_3d | mask_dist).astype(f32)
    um8 = jnp.broadcast_to(um[:, None], (N, 8))

    h_add, v_out = pl.pallas_call(
        _final_body,
        grid=(N // NT,),
        in_specs=[
            pl.BlockSpec((NT, H1), lambda i: (i, 0)),
            pl.BlockSpec((NT, 3 * H1), lambda i: (i, 0)),
            _w((H1, H1)),
            pl.BlockSpec((NT, 8), lambda i: (i, 0)),
        ],
        out_specs=[pl.BlockSpec((NT, H1), lambda i: (i, 0)),
                   pl.BlockSpec((NT, 3 * H1), lambda i: (i, 0))],
        out_shape=[jax.ShapeDtypeStruct((N, H1), f32),
                   jax.ShapeDtypeStruct((N, 3 * H1), f32)],
    )(agg, vsum, W_h, um8)

    return (h_add,
            jnp.transpose(v_out.reshape(N, 3, H1), (0, 2, 1)))


# speedup vs baseline: 9.6413x; 1.7419x over previous
import functools
import jax
import jax.numpy as jnp
import numpy as np
from jax.experimental import pallas as pl
from jax.experimental.pallas import tpu as pltpu
from jax.experimental.pallas import tpu_sc as plsc

CUTOFF = 6.0
NRBF = 64


def _silu(x):
    return x * jax.nn.sigmoid(x)


def _rbf(d_col, tile):
    centers = jax.lax.broadcasted_iota(jnp.int32, (tile, NRBF), 1).astype(
        jnp.float32) * (CUTOFF / (NRBF - 1))
    width = CUTOFF / NRBF
    gamma = 1.0 / (2.0 * width * width)
    env = 0.5 * (jnp.cos(jnp.pi * jnp.clip(d_col / CUTOFF, 0.0, 1.0)) + 1.0)
    return jnp.exp(-gamma * (d_col - centers) ** 2) * env


def _nodeproj_body(h_ref, h2d_ref, vn_ref, wi_ref, w1_ref,
                   ps_ref, pd_ref, psv_ref):
    f32 = jnp.float32
    a = jnp.dot(h_ref[...], wi_ref[0:128, :], preferred_element_type=f32)
    br = jnp.dot(h2d_ref[...], wi_ref[128:256, :], preferred_element_type=f32)
    cvec = jnp.dot(vn_ref[...], wi_ref[128:256, :], preferred_element_type=f32)
    nr = a + br
    nv = a + cvec
    w1s = w1_ref[0:128, :]
    w1d = w1_ref[128:256, :]
    ps_ref[...] = jnp.dot(nr, w1s, preferred_element_type=f32)
    pd_ref[...] = jnp.dot(nr, w1d, preferred_element_type=f32)
    psv_ref[...] = jnp.dot(nv, w1s, preferred_element_type=f32)


def _dense3d_body(psv_ref, pd_ref, geo_ref, et_ref, w1_ref, b1_ref,
                  w2_ref, b2_ref, wa_ref, m3_ref, v3_ref):
    f32 = jnp.float32
    tile = psv_ref.shape[0]
    d = geo_ref[:, 0:1]
    rbf = _rbf(d, tile)
    bc1 = b1_ref[...] + jnp.dot(et_ref[1:2, :], w1_ref[320:336, :],
                                preferred_element_type=f32)
    pre = psv_ref[...] + pd_ref[...] + jnp.dot(
        rbf, w1_ref[336:400, :], preferred_element_type=f32) + bc1
    m1 = _silu(pre)
    m = _silu(jnp.dot(m1, w2_ref[...], preferred_element_type=f32)
              + b2_ref[...])
    msk = geo_ref[:, 4:5]
    m3_ref[...] = m * msk
    a = jnp.dot(m, wa_ref[...], preferred_element_type=f32) * msk
    for k in range(3):
        v3_ref[:, pl.ds(k * 128, 128)] = a * geo_ref[:, 1 + k:2 + k]


def _edge_body(n2tiles, srcr_ref, dstr_ref, attr_ref, aux_ref, et_ref,
               w_e2d_ref, w_rbf_ref, w1_ref, b1_ref, w2_ref, b2_ref,
               wa_ref, wg_ref, m_ref, *vec_refs):
    f32 = jnp.float32
    tile = attr_ref.shape[0]
    i = pl.program_id(0)
    w1ea = w1_ref[256:320, :]
    bc = b1_ref[...] + jnp.dot(
        jnp.where(i < n2tiles, et_ref[0:1, :], et_ref[2:3, :]),
        w1_ref[320:336, :], preferred_element_type=f32)
    m2d = jnp.dot(w_e2d_ref[...], w1ea, preferred_element_type=f32)
    t2 = jnp.dot(attr_ref[...], m2d, preferred_element_type=f32)
    mdist = jnp.dot(w_rbf_ref[...], w1ea, preferred_element_type=f32)
    rbv = _rbf(aux_ref[:, 0:1], tile)
    td = jnp.dot(rbv, mdist, preferred_element_type=f32)
    t = jnp.where(i < n2tiles, t2, td) + bc
    dx = srcr_ref[:, 512:513] - dstr_ref[:, 128:129]
    dy = srcr_ref[:, 513:514] - dstr_ref[:, 129:130]
    dz = srcr_ref[:, 514:515] - dstr_ref[:, 130:131]
    d = jnp.sqrt(dx * dx + dy * dy + dz * dz + 1e-8)
    rbf = _rbf(d, tile)
    pre = (srcr_ref[:, 0:128] + dstr_ref[:, 0:128] + t
           + jnp.dot(rbf, w1_ref[336:400, :], preferred_element_type=f32))
    m1 = _silu(pre)
    m = _silu(jnp.dot(m1, w2_ref[...], preferred_element_type=f32)
              + b2_ref[...])
    m_ref[...] = m
    a = jnp.dot(m, wa_ref[...], preferred_element_type=f32)
    g = jnp.dot(m, wg_ref[...], preferred_element_type=f32)
    dinv = 1.0 / (d + 1.0)
    for k, (vref, dk) in enumerate(zip(vec_refs, (dx, dy, dz))):
        vref[...] = (a * (dk * dinv)
                     + g * srcr_ref[:, pl.ds(128 + k * 128, 128)])


def _final_body(agg_ref, vsum_ref, wh_ref, um_ref, h_ref, v_ref):
    f32 = jnp.float32
    um = um_ref[:, 0:1]
    h = jnp.dot(agg_ref[...], wh_ref[...], preferred_element_type=f32)
    h_ref[...] = jnp.clip(h, -100.0, 100.0) * um
    v_ref[...] = jnp.clip(vsum_ref[...], -100.0, 100.0) * um


def _make_sc_scatter(E_s, D, B, NB, NPC, R):
    mesh = plsc.VectorSubcoreMesh(core_axis_name="c", subcore_axis_name="s",
                                  num_cores=2, num_subcores=16)
    Z = R // 16
    ZC = 32
    nzch = Z // ZC
    nj = B // 128
    k16 = D // 16
    chunk = B * NB
    crows = chunk // 128

    def body(data_hbm, dst_hbm, out_hbm, databuf, idxbuf, zbuf, acc):
        c = jax.lax.axis_index("c")
        s = jax.lax.axis_index("s")

        def zstore(i, car):
            r = i // k16
            col = (i % k16) * 16
            zbuf[r, pl.ds(col, 16)] = jnp.zeros((16,), jnp.float32)
            return car
        jax.lax.fori_loop(0, ZC * k16, zstore, 0)
        for t in range(nzch):
            off = pl.multiple_of(s * Z + t * ZC, ZC)
            pltpu.sync_copy(zbuf, acc.at[pl.ds(off, ZC)])
        plsc.subcore_barrier()

        ibase = pl.multiple_of(s * crows, 8)
        pltpu.sync_copy(dst_hbm.at[pl.ds(ibase, crows)], idxbuf)
        lo = c * NPC

        def fix(i, car2):
            j = i // 8
            col = (i % 8) * 16
            v = idxbuf[j, pl.ds(col, 16)]
            loc = v - lo
            ok = (loc >= 0) & (loc < NPC)
            idxbuf[j, pl.ds(col, 16)] = jnp.where(ok, loc, NPC)
            return car2
        jax.lax.fori_loop(0, crows * 8, fix, 0)

        def block(b, car):
            base = pl.multiple_of(s * chunk + b * B, 8)
            pltpu.sync_copy(data_hbm.at[pl.ds(base, B)], databuf)
            for j in range(nj):
                pltpu.sync_copy(databuf.at[pl.ds(j * 128, 128)],
                                acc.at[idxbuf.at[b * nj + j]], add=True)
            return car
        jax.lax.fori_loop(0, NB, block, 0)
        plsc.subcore_barrier()
        for t in range(nzch):
            off = pl.multiple_of(s * Z + t * ZC, ZC)
            pltpu.sync_copy(acc.at[pl.ds(off, ZC)],
                            out_hbm.at[c, pl.ds(off, ZC)])

    f32 = jnp.float32
    return pl.kernel(
        body,
        out_type=jax.ShapeDtypeStruct((2, R, D), f32),
        mesh=mesh,
        scratch_types=[pltpu.VMEM((B, D), f32),
                       pltpu.VMEM((crows, 128), jnp.int32),
                       pltpu.VMEM((ZC, D), f32),
                       pltpu.VMEM_SHARED((R, D), f32)],
    )


def _make_sc_gather(E_s, DS, DD):
    mesh = plsc.VectorSubcoreMesh(core_axis_name="c", subcore_axis_name="s",
                                  num_cores=2, num_subcores=16)
    chunk = E_s // 32
    crows = chunk // 128

    def body(tsrc_hbm, tdst_hbm, src_hbm, dst_hbm, osrc_hbm, odst_hbm,
             idxbuf, srcbuf, dstbuf):
        c = jax.lax.axis_index("c")
        s = jax.lax.axis_index("s")
        w = c * 16 + s
        ibase = pl.multiple_of(w * crows, 8)

        pltpu.sync_copy(src_hbm.at[pl.ds(ibase, crows)], idxbuf)

        def ga(j, car):
            base = pl.multiple_of(w * chunk + j * 128, 8)
            pltpu.sync_copy(tsrc_hbm.at[idxbuf.at[j]], srcbuf)
            pltpu.sync_copy(srcbuf, osrc_hbm.at[pl.ds(base, 128)])
            return car
        jax.lax.fori_loop(0, crows, ga, 0)

        pltpu.sync_copy(dst_hbm.at[pl.ds(ibase, crows)], idxbuf)

        def gb(j, car):
            base = pl.multiple_of(w * chunk + j * 128, 8)
            pltpu.sync_copy(tdst_hbm.at[idxbuf.at[j]], dstbuf)
            pltpu.sync_copy(dstbuf, odst_hbm.at[pl.ds(base, 128)])
            return car
        jax.lax.fori_loop(0, crows, gb, 0)

    f32 = jnp.float32
    return pl.kernel(
        body,
        out_type=[jax.ShapeDtypeStruct((E_s, DS), f32),
                  jax.ShapeDtypeStruct((E_s, DD), f32)],
        mesh=mesh,
        scratch_types=[pltpu.VMEM((crows, 128), jnp.int32),
                       pltpu.VMEM((128, DS), f32),
                       pltpu.VMEM((128, DD), f32)],
    )


def _w(spec_shape):
    return pl.BlockSpec(spec_shape, lambda i: tuple(0 for _ in spec_shape))


def kernel(H, V, Z, layer_i, H_2d, mask_2d, E_2d_index, E_2d_attr, Z_3d,
           mask_3d, E_dist_index, E_dist_val, virtual_node_embed,
           edge_type_table, W_rbf, W_edge2d, W_i, W1, b1, W2, b2, W_h,
           W_a, W_g):
    del layer_i
    f32 = jnp.float32
    N = H.shape[0]
    E1 = E_2d_index.shape[1]
    E2 = E_dist_index.shape[1]
    H1 = H.shape[1]

    b1r = b1.reshape(1, H1)
    b2r = b2.reshape(1, H1)

    NT = 2000
    ps, pd, psv = pl.pallas_call(
        _nodeproj_body,
        grid=(N // NT,),
        in_specs=[
            pl.BlockSpec((NT, H1), lambda i: (i, 0)),
            pl.BlockSpec((NT, H1), lambda i: (i, 0)),
            _w((1, H1)), _w((2 * H1, H1)), _w((400, H1)),
        ],
        out_specs=[pl.BlockSpec((NT, H1), lambda i: (i, 0))] * 3,
        out_shape=[jax.ShapeDtypeStruct((N, H1), f32)] * 3,
    )(H, H_2d, virtual_node_embed, W_i, W1)

    rel3 = Z_3d - Z
    d3 = jnp.sqrt(jnp.sum(rel3 * rel3, axis=-1) + 1e-8)
    ru3 = rel3 / (d3[:, None] + 1.0)
    m3f = mask_3d.astype(f32)
    geo3 = jnp.concatenate(
        [d3[:, None], ru3, m3f[:, None], jnp.zeros((N, 3), f32)], axis=1)
    m3, v3 = pl.pallas_call(
        _dense3d_body,
        grid=(N // NT,),
        in_specs=[
            pl.BlockSpec((NT, H1), lambda i: (i, 0)),
            pl.BlockSpec((NT, H1), lambda i: (i, 0)),
            pl.BlockSpec((NT, 8), lambda i: (i, 0)),
            _w((3, 16)), _w((400, H1)), _w((1, H1)),
            _w((H1, H1)), _w((1, H1)), _w((H1, H1)),
        ],
        out_specs=[pl.BlockSpec((NT, H1), lambda i: (i, 0)),
                   pl.BlockSpec((NT, 3 * H1), lambda i: (i, 0))],
        out_shape=[jax.ShapeDtypeStruct((N, H1), f32),
                   jax.ShapeDtypeStruct((N, 3 * H1), f32)],
    )(psv, pd, geo3, edge_type_table, W1, b1r, W2, b2r, W_a)

    ET = 256
    e1p = (E1 + ET - 1) // ET * ET
    ep = ((e1p + E2 + 8191) // 8192) * 8192
    n2tiles = e1p // ET

    def pad_idx(x, n, fill):
        return jnp.concatenate([x, jnp.full((n - x.shape[0],), fill, x.dtype)])

    src = jnp.concatenate([pad_idx(E_2d_index[0], e1p, 0),
                           pad_idx(E_dist_index[0], ep - e1p, 0)])
    dst_g = jnp.concatenate([pad_idx(E_2d_index[1], e1p, 0),
                             pad_idx(E_dist_index[1], ep - e1p, 0)])
    dst_s = jnp.concatenate([pad_idx(E_2d_index[1], e1p, N),
                             pad_idx(E_dist_index[1], ep - e1p, N)])

    dval = jnp.concatenate([jnp.zeros((e1p,), f32),
                            pad_idx(E_dist_val, ep - e1p, 0.0)])
    aux = jnp.pad(dval[:, None], ((0, 0), (0, 7)))
    attr = jnp.zeros((ep, 16), f32).at[:E1].set(E_2d_attr.T)
    vt = jnp.transpose(V, (0, 2, 1)).reshape(N, 3 * H1)

    tbl_src = jnp.concatenate(
        [ps, vt, Z, jnp.zeros((N, 125), f32)], axis=1)
    tbl_dst = jnp.concatenate(
        [pd, Z, jnp.zeros((N, 125), f32)], axis=1)
    src_rows, dst_rows = _make_sc_gather(ep, 640, 256)(
        tbl_src, tbl_dst, src.reshape(ep // 128, 128),
        dst_g.reshape(ep // 128, 128))

    m_e = pl.pallas_call(
        functools.partial(_edge_body, n2tiles),
        grid=(ep // ET,),
        in_specs=[
            pl.BlockSpec((ET, 640), lambda i: (i, 0)),
            pl.BlockSpec((ET, 256), lambda i: (i, 0)),
            pl.BlockSpec((ET, 16), lambda i: (i, 0)),
            pl.BlockSpec((ET, 8), lambda i: (i, 0)),
            _w((3, 16)), _w((16, 64)), _w((64, 64)), _w((400, H1)),
            _w((1, H1)), _w((H1, H1)), _w((1, H1)), _w((H1, H1)),
            _w((H1, H1)),
        ],
        out_specs=[pl.BlockSpec((ET, H1), lambda i: (i, 0))] * 4,
        out_shape=[jax.ShapeDtypeStruct((ep, H1), f32)] * 4,
    )(src_rows, dst_rows, attr, aux, edge_type_table, W_edge2d, W_rbf,
      W1, b1r, W2, b2r, W_a, W_g)
    m_e, vx_e, vy_e, vz_e = m_e

    NPC = N // 2
    R = ((NPC + 1 + 1023) // 1024) * 1024
    dst2d = dst_s.reshape(ep // 128, 128)
    scat = _make_sc_scatter(ep, H1, 512, ep // (16 * 512), NPC, R)

    def seg(x):
        o = scat(x, dst2d)
        return jnp.concatenate([o[0, :NPC], o[1, :NPC]])

    agg = seg(m_e) + m3
    vsum = jnp.concatenate([seg(vx_e), seg(vy_e), seg(vz_e)], axis=1) + v3

    mask_dist = (jnp.zeros((N,), bool).at[E_dist_index[0]].set(True)
                 .at[E_dist_index[1]].set(True))
    um = (mask_2d | mask_3d | mask_dist).astype(f32)
    um8 = jnp.broadcast_to(um[:, None], (N, 8))

    h_add, v_out = pl.pallas_call(
        _final_body,
        grid=(N // NT,),
        in_specs=[
            pl.BlockSpec((NT, H1), lambda i: (i, 0)),
            pl.BlockSpec((NT, 3 * H1), lambda i: (i, 0)),
            _w((H1, H1)),
            pl.BlockSpec((NT, 8), lambda i: (i, 0)),
        ],
        out_specs=[pl.BlockSpec((NT, H1), lambda i: (i, 0)),
                   pl.BlockSpec((NT, 3 * H1), lambda i: (i, 0))],
        out_shape=[jax.ShapeDtypeStruct((N, H1), f32),
                   jax.ShapeDtypeStruct((N, 3 * H1), f32)],
    )(agg, vsum, W_h, um8)

    return (h_add,
            jnp.transpose(v_out.reshape(N, 3, H1), (0, 2, 1)))
